# baseline (speedup 1.0000x reference)
import functools

import jax
import jax.numpy as jnp
from jax import lax
from jax.experimental import pallas as pl
from jax.experimental.pallas import tpu as pltpu
from jax.experimental.pallas import tpu_sc as plsc

_ROWS = 50000
_D = 128
_NW = 32
_PW = 1568
_C = 56
_NCH = _PW // _C
_NB = 8
_LAST_W_BASE = _ROWS - _PW
_LOOP_LO = _NB
_LOOP_HI = _NCH - ((_NCH - _NB) % _NB)


@functools.partial(
    pl.kernel,
    mesh=plsc.VectorSubcoreMesh(core_axis_name="c", subcore_axis_name="s"),
    out_type=jax.ShapeDtypeStruct((_ROWS, _D), jnp.float32),
    scratch_types=[
        pltpu.VMEM((_PW,), jnp.int32),
        pltpu.VMEM((_NB, _C, _D), jnp.float32),
    ] + [pltpu.SemaphoreType.DMA] * (2 * _NB),
)
def _gather_sc(table_hbm, idx_hbm, out_hbm, idx_v, bufs, *sems):
    sgs = sems[:_NB]
    sss = sems[_NB:]
    wid = lax.axis_index("s") * 2 + lax.axis_index("c")
    base_w = jnp.minimum(wid * _PW, _LAST_W_BASE)
    _HEAD = _NB * _C
    pltpu.sync_copy(idx_hbm.at[pl.ds(base_w, _HEAD)], idx_v.at[pl.ds(0, _HEAD)])

    def start_gather(j, p):
        pltpu.async_copy(
            table_hbm.at[idx_v.at[pl.ds(j * _C, _C)]], bufs.at[p], sgs[p])

    def start_store(j, p):
        pltpu.async_copy(
            bufs.at[p], out_hbm.at[pl.ds(base_w + j * _C, _C)], sss[p])

    def wait_gather(p):
        pltpu.make_async_copy(
            out_hbm.at[pl.ds(base_w, _C)], bufs.at[p], sgs[p]).wait()

    def wait_store(p):
        pltpu.make_async_copy(
            bufs.at[p], out_hbm.at[pl.ds(base_w, _C)], sss[p]).wait()

    for b in range(_NB):
        start_gather(b, b)
    pltpu.sync_copy(idx_hbm.at[pl.ds(base_w + _HEAD, _PW - _HEAD)],
                    idx_v.at[pl.ds(_HEAD, _PW - _HEAD)])
    wait_gather(0)
    start_store(0, 0)

    @pl.loop(_LOOP_LO, _LOOP_HI, step=_NB)
    def _body(j0):
        for b in range(_NB):
            j = j0 + b
            p = b
            q = (b + 1) % _NB
            wait_store(p)
            start_gather(j, p)
            wait_gather(q)
            start_store(j - (_NB - 1), q)

    for j in range(_LOOP_HI, _NCH):
        p = j % _NB
        q = (j + 1) % _NB
        wait_store(p)
        start_gather(j, p)
        wait_gather(q)
        start_store(j - (_NB - 1), q)

    for j in range(_NCH - (_NB - 1), _NCH):
        p = j % _NB
        wait_gather(p)
        start_store(j, p)
    for p in range(_NB):
        wait_store(p)


def kernel(features, sel_idx_up):
    idx = sel_idx_up.reshape(-1)
    return _gather_sc(features, idx)


# speedup vs baseline: 1.0171x; 1.0171x over previous
import functools

import jax
import jax.numpy as jnp
from jax import lax
from jax.experimental import pallas as pl
from jax.experimental.pallas import tpu as pltpu
from jax.experimental.pallas import tpu_sc as plsc

_ROWS = 50000
_D = 128
_NW = 32
_PW = 1568
_C = 56
_NCH = _PW // _C
_NB = 8
_LAST_W_BASE = _ROWS - _PW
_LOOP_LO = _NB
_LOOP_HI = _NCH - ((_NCH - _NB) % _NB)


@functools.partial(
    pl.kernel,
    mesh=plsc.VectorSubcoreMesh(core_axis_name="c", subcore_axis_name="s"),
    out_type=jax.ShapeDtypeStruct((_ROWS, _D), jnp.float32),
    scratch_types=[
        pltpu.VMEM((_PW,), jnp.int32),
        pltpu.VMEM((_NB, _C, _D), jnp.float32),
    ] + [pltpu.SemaphoreType.DMA] * (2 * _NB),
)
def _gather_sc(table_hbm, idx_hbm, out_hbm, idx_v, bufs, *sems):
    sgs = sems[:_NB]
    sss = sems[_NB:]
    wid = lax.axis_index("s") * 2 + lax.axis_index("c")
    base_w = jnp.minimum(wid * _PW, _LAST_W_BASE)
    pltpu.sync_copy(idx_hbm.at[pl.ds(base_w, _PW)], idx_v)

    def start_gather(j, p):
        pltpu.async_copy(
            table_hbm.at[idx_v.at[pl.ds(j * _C, _C)]], bufs.at[p], sgs[p])

    def start_store(j, p):
        pltpu.async_copy(
            bufs.at[p], out_hbm.at[pl.ds(base_w + j * _C, _C)], sss[p])

    def wait_gather(p):
        pltpu.make_async_copy(
            out_hbm.at[pl.ds(base_w, _C)], bufs.at[p], sgs[p]).wait()

    def wait_store(p):
        pltpu.make_async_copy(
            bufs.at[p], out_hbm.at[pl.ds(base_w, _C)], sss[p]).wait()

    for b in range(_NB):
        start_gather(b, b)
    wait_gather(0)
    start_store(0, 0)

    @pl.loop(_LOOP_LO, _LOOP_HI, step=_NB)
    def _body(j0):
        for b in range(_NB):
            j = j0 + b
            p = b
            q = (b + 1) % _NB
            wait_store(p)
            start_gather(j, p)
            wait_gather(q)
            start_store(j - (_NB - 1), q)

    for j in range(_LOOP_HI, _NCH):
        p = j % _NB
        q = (j + 1) % _NB
        wait_store(p)
        start_gather(j, p)
        wait_gather(q)
        start_store(j - (_NB - 1), q)

    for j in range(_NCH - (_NB - 1), _NCH):
        p = j % _NB
        wait_gather(p)
        start_store(j, p)
    for p in range(_NB):
        wait_store(p)


def kernel(features, sel_idx_up):
    idx = sel_idx_up.reshape(-1)
    return _gather_sc(features, idx)
